# baseline (speedup 1.0000x reference)
import jax
import jax.numpy as jnp
from jax.experimental import pallas as pl
from jax.experimental.pallas import tpu as pltpu

_N = 10000
_IN_FT = 256
_OUT_FT = 256
_BM = 200
_NCHUNKS = _N // _BM
_NBUF = 4


def _chunk_copy(adj_ref, bufs_ref, sems_ref, chunk):
    slot = jax.lax.rem(chunk, _NBUF)
    return pltpu.make_async_copy(
        adj_ref.at[pl.ds(chunk * _BM, _BM), :],
        bufs_ref.at[slot],
        sems_ref.at[slot],
    )


def _gcn_kernel(seq_ref, wt_ref, bfc_ref, adj_ref, bias_ref, ap_ref,
                out_ref, sf_ref, bufs_ref, sems_ref):
    i = pl.program_id(0)

    @pl.when(i == 0)
    def _prologue():
        for c in range(_NBUF - 1):
            _chunk_copy(adj_ref, bufs_ref, sems_ref, c).start()
        sf_ref[...] = (
            jnp.dot(seq_ref[...], wt_ref[...],
                    preferred_element_type=jnp.float32)
            + bfc_ref[...]
        )

    _chunk_copy(adj_ref, bufs_ref, sems_ref, i).wait()
    a = bufs_ref[jax.lax.rem(i, _NBUF)]
    acc = jnp.dot(a, sf_ref[...], preferred_element_type=jnp.float32)
    acc = acc + bias_ref[...]
    out_ref[...] = jnp.where(acc >= 0.0, acc, ap_ref[0, 0] * acc)

    nxt = i + _NBUF - 1
    @pl.when(nxt < _NCHUNKS)
    def _prefetch():
        _chunk_copy(adj_ref, bufs_ref, sems_ref, nxt).start()


def kernel(seq, adj, W_fc, b_fc, bias, a_prelu):
    wt = W_fc.T
    bfc2 = b_fc.reshape(1, _OUT_FT)
    bias2 = bias.reshape(1, _OUT_FT)
    ap2 = a_prelu.reshape(1, 1)

    return pl.pallas_call(
        _gcn_kernel,
        grid=(_NCHUNKS,),
        in_specs=[
            pl.BlockSpec((_N, _IN_FT), lambda i: (0, 0)),
            pl.BlockSpec((_IN_FT, _OUT_FT), lambda i: (0, 0)),
            pl.BlockSpec((1, _OUT_FT), lambda i: (0, 0)),
            pl.BlockSpec(memory_space=pl.ANY),
            pl.BlockSpec((1, _OUT_FT), lambda i: (0, 0)),
            pl.BlockSpec((1, 1), lambda i: (0, 0)),
        ],
        out_specs=pl.BlockSpec((_BM, _OUT_FT), lambda i: (i, 0)),
        out_shape=jax.ShapeDtypeStruct((_N, _OUT_FT), jnp.float32),
        scratch_shapes=[
            pltpu.VMEM((_N, _OUT_FT), jnp.float32),
            pltpu.VMEM((_NBUF, _BM, _N), jnp.float32),
            pltpu.SemaphoreType.DMA((_NBUF,)),
        ],
        compiler_params=pltpu.CompilerParams(
            dimension_semantics=("arbitrary",),
        ),
    )(seq, wt, bfc2, adj, bias2, ap2)
